# Initial kernel scaffold; baseline (speedup 1.0000x reference)
import jax
import jax.numpy as jnp
from jax.experimental import pallas as pl


def kernel(atoms_xyz, atom_type, atom_i_idx, atom_j_idx, dist_ij, embed_table, W, b):
    raise NotImplementedError("write your pallas kernel here")


# speedup vs baseline: 1.7789x; 1.7789x over previous
import functools

import jax
import jax.numpy as jnp
from jax import lax
from jax.experimental import pallas as pl
from jax.experimental.pallas import tpu as pltpu
from jax.experimental.pallas import tpu_sc as plsc

_N_ATOMS = 100000
_N_CENTER = 8192
_N_NEIGH = 8
_N_TYPES = 16
_TBL_W = 16
_NC = 2
_NS = 16
_NW = _NC * _NS
_CBLK = 128


def _sc_gather(table, idx_j, idx_i):
    nj = idx_j.shape[0] // _NW
    ni = idx_i.shape[0] // _NW
    mesh = plsc.VectorSubcoreMesh(core_axis_name="c", subcore_axis_name="s")

    @functools.partial(
        pl.kernel,
        mesh=mesh,
        compiler_params=pltpu.CompilerParams(use_tc_tiling_on_sc=False),
        out_type=[
            jax.ShapeDtypeStruct((idx_j.shape[0], _TBL_W), jnp.float32),
            jax.ShapeDtypeStruct((idx_i.shape[0], _TBL_W), jnp.float32),
        ],
        scratch_types=[
            pltpu.VMEM((nj,), jnp.int32),
            pltpu.VMEM((nj, _TBL_W), jnp.float32),
            pltpu.VMEM((ni,), jnp.int32),
            pltpu.VMEM((ni, _TBL_W), jnp.float32),
            pltpu.SemaphoreType.DMA,
        ],
    )
    def k(table_hbm, idxj_hbm, idxi_hbm, outj_hbm, outi_hbm,
          idxj_v, rowsj_v, idxi_v, rowsi_v, sem):
        wid = lax.axis_index("s") * _NC + lax.axis_index("c")
        bj = wid * nj
        pltpu.sync_copy(idxj_hbm.at[pl.ds(bj, nj)], idxj_v)
        pltpu.async_copy(table_hbm.at[idxj_v], rowsj_v, sem).wait()
        pltpu.sync_copy(rowsj_v, outj_hbm.at[pl.ds(bj, nj)])
        bi = wid * ni
        pltpu.sync_copy(idxi_hbm.at[pl.ds(bi, ni)], idxi_v)
        pltpu.async_copy(table_hbm.at[idxi_v], rowsi_v, sem).wait()
        pltpu.sync_copy(rowsi_v, outi_hbm.at[pl.ds(bi, ni)])

    return k(table, idx_j, idx_i)


def _tc_body(xs_ref, ys_ref, zs_ref, tj_ref, ti_ref, dist_ref,
             w_ref, e_ref, b_ref, out_ref):
    xs = xs_ref[...]
    ys = ys_ref[...]
    zs = zs_ref[...]
    tj = tj_ref[...]
    ti = ti_ref[...]
    dist = dist_ref[...]
    w = w_ref[...]
    e = e_ref[...]
    b = b_ref[...]
    c = xs.shape[0]

    w0 = w[0:1, :].reshape(1, 1, 64)
    w1 = w[1:2, :].reshape(1, 1, 64)
    w2 = w[2:3, :].reshape(1, 1, 64)
    ew1 = jnp.dot(e, w[3:19, :], preferred_element_type=jnp.float32)
    ew2 = jnp.dot(e, w[19:35, :], preferred_element_type=jnp.float32)
    ew3 = jnp.dot(e, w[35:51, :], preferred_element_type=jnp.float32)

    iota3 = lax.broadcasted_iota(jnp.int32, (c, _N_NEIGH, _N_TYPES), 2)
    onej = (tj.astype(jnp.int32) == iota3).astype(jnp.float32)
    onej2 = onej.reshape(c * _N_NEIGH, _N_TYPES)
    iota2 = lax.broadcasted_iota(jnp.int32, (c, _N_TYPES), 1)
    onei = (ti.astype(jnp.int32) == iota2).astype(jnp.float32)

    inv_d = 1.0 / dist
    m2 = jnp.dot(onej2, ew2, preferred_element_type=jnp.float32)
    m3 = jnp.dot(onej2, ew3, preferred_element_type=jnp.float32)
    a3 = dist * w0 + inv_d * m2.reshape(c, _N_NEIGH, 64)
    b3 = dist * w1 + inv_d * m3.reshape(c, _N_NEIGH, 64)
    cc = (jnp.dot(onei, ew1, preferred_element_type=jnp.float32)
          + b).reshape(c, 1, 64)

    for j in range(_N_NEIGH):
        dx = xs[:, j:j + 1, :] - xs
        dy = ys[:, j:j + 1, :] - ys
        dz = zs[:, j:j + 1, :] - zs
        djk = jnp.sqrt(dx * dx + dy * dy + dz * dz)
        dij = dist[:, j:j + 1, :]
        mx = jnp.maximum(dij, dist)
        mn = jnp.minimum(dij, dist)
        djkn = (djk - mx + mn) / (2.0 * mn)
        if j == 0:
            djk7 = djkn[:, 1:, :]
            b7 = b3[:, 1:, :]
        elif j == _N_NEIGH - 1:
            djk7 = djkn[:, :j, :]
            b7 = b3[:, :j, :]
        else:
            djk7 = jnp.concatenate([djkn[:, :j, :], djkn[:, j + 1:, :]], axis=1)
            b7 = jnp.concatenate([b3[:, :j, :], b3[:, j + 1:, :]], axis=1)
        base = cc + a3[:, j:j + 1, :]
        out_ref[:, j, :, :] = base + b7 + djk7 * w2


def kernel(atoms_xyz, atom_type, atom_i_idx, atom_j_idx, dist_ij, embed_table, W, b):
    n_atoms = atoms_xyz.shape[0]
    table = jnp.concatenate(
        [atoms_xyz.astype(jnp.float32),
         atom_type.astype(jnp.float32)[:, None],
         jnp.zeros((n_atoms, _TBL_W - 4), jnp.float32)], axis=1)
    idx_j = atom_j_idx.reshape(-1).astype(jnp.int32)
    idx_i = atom_i_idx.reshape(-1).astype(jnp.int32)

    gj_flat, gi = _sc_gather(table, idx_j, idx_i)
    gj = gj_flat.reshape(_N_CENTER, _N_NEIGH, _TBL_W)
    xs = gj[:, :, 0:1]
    ys = gj[:, :, 1:2]
    zs = gj[:, :, 2:3]
    tj = gj[:, :, 3:4]
    ti = gi[:, 3:4]
    dist3 = dist_ij.astype(jnp.float32)[:, :, None]
    b2 = b.astype(jnp.float32).reshape(1, 64)

    grid = _N_CENTER // _CBLK
    out4 = pl.pallas_call(
        _tc_body,
        grid=(grid,),
        in_specs=[
            pl.BlockSpec((_CBLK, _N_NEIGH, 1), lambda i: (i, 0, 0)),
            pl.BlockSpec((_CBLK, _N_NEIGH, 1), lambda i: (i, 0, 0)),
            pl.BlockSpec((_CBLK, _N_NEIGH, 1), lambda i: (i, 0, 0)),
            pl.BlockSpec((_CBLK, _N_NEIGH, 1), lambda i: (i, 0, 0)),
            pl.BlockSpec((_CBLK, 1), lambda i: (i, 0)),
            pl.BlockSpec((_CBLK, _N_NEIGH, 1), lambda i: (i, 0, 0)),
            pl.BlockSpec((51, 64), lambda i: (0, 0)),
            pl.BlockSpec((16, 16), lambda i: (0, 0)),
            pl.BlockSpec((1, 64), lambda i: (0, 0)),
        ],
        out_specs=pl.BlockSpec((_CBLK, _N_NEIGH, _N_NEIGH - 1, 64),
                               lambda i: (i, 0, 0, 0)),
        out_shape=jax.ShapeDtypeStruct(
            (_N_CENTER, _N_NEIGH, _N_NEIGH - 1, 64), jnp.float32),
    )(xs, ys, zs, tj, ti, dist3, W.astype(jnp.float32),
      embed_table.astype(jnp.float32), b2)
    return out4.reshape(_N_CENTER, _N_NEIGH * (_N_NEIGH - 1), 64)


# speedup vs baseline: 3.3216x; 1.8672x over previous
import functools

import jax
import jax.numpy as jnp
from jax import lax
from jax.experimental import pallas as pl
from jax.experimental.pallas import tpu as pltpu
from jax.experimental.pallas import tpu_sc as plsc

_N_ATOMS = 100000
_N_CENTER = 8192
_N_NEIGH = 8
_N_TYPES = 16
_NC = 2
_NS = 16
_NW = _NC * _NS
_CBLK = 128


def _sc_gather(xf, yf, zf, tf, idx_j, idx_i):
    nj = idx_j.shape[0] // _NW
    ni = idx_i.shape[0] // _NW
    mesh = plsc.VectorSubcoreMesh(core_axis_name="c", subcore_axis_name="s")
    n_total = idx_j.shape[0]

    @functools.partial(
        pl.kernel,
        mesh=mesh,
        compiler_params=pltpu.CompilerParams(use_tc_tiling_on_sc=False),
        out_type=[
            jax.ShapeDtypeStruct((n_total,), jnp.float32),
            jax.ShapeDtypeStruct((n_total,), jnp.float32),
            jax.ShapeDtypeStruct((n_total,), jnp.float32),
            jax.ShapeDtypeStruct((n_total,), jnp.float32),
            jax.ShapeDtypeStruct((idx_i.shape[0],), jnp.float32),
        ],
        scratch_types=[
            pltpu.VMEM((nj,), jnp.int32),
            pltpu.VMEM((nj,), jnp.float32),
            pltpu.VMEM((nj,), jnp.float32),
            pltpu.VMEM((nj,), jnp.float32),
            pltpu.VMEM((nj,), jnp.float32),
            pltpu.VMEM((ni,), jnp.int32),
            pltpu.VMEM((ni,), jnp.float32),
            pltpu.SemaphoreType.DMA,
        ],
    )
    def k(xf_hbm, yf_hbm, zf_hbm, tf_hbm, idxj_hbm, idxi_hbm,
          xg_hbm, yg_hbm, zg_hbm, tg_hbm, tig_hbm,
          idxj_v, xv, yv, zv, tv, idxi_v, tiv, sem):
        wid = lax.axis_index("s") * _NC + lax.axis_index("c")
        bj = wid * nj
        pltpu.sync_copy(idxj_hbm.at[pl.ds(bj, nj)], idxj_v)
        cx = pltpu.async_copy(xf_hbm.at[idxj_v], xv, sem)
        cy = pltpu.async_copy(yf_hbm.at[idxj_v], yv, sem)
        cz = pltpu.async_copy(zf_hbm.at[idxj_v], zv, sem)
        ct = pltpu.async_copy(tf_hbm.at[idxj_v], tv, sem)
        cx.wait()
        cy.wait()
        cz.wait()
        ct.wait()
        pltpu.sync_copy(xv, xg_hbm.at[pl.ds(bj, nj)])
        pltpu.sync_copy(yv, yg_hbm.at[pl.ds(bj, nj)])
        pltpu.sync_copy(zv, zg_hbm.at[pl.ds(bj, nj)])
        pltpu.sync_copy(tv, tg_hbm.at[pl.ds(bj, nj)])
        bi = wid * ni
        pltpu.sync_copy(idxi_hbm.at[pl.ds(bi, ni)], idxi_v)
        pltpu.async_copy(tf_hbm.at[idxi_v], tiv, sem).wait()
        pltpu.sync_copy(tiv, tig_hbm.at[pl.ds(bi, ni)])

    return k(xf, yf, zf, tf, idx_j, idx_i)


def _tc_body(xT_ref, yT_ref, zT_ref, dT_ref, tjT_ref, ti_ref,
             w_ref, e_ref, b_ref, out_ref):
    xT = xT_ref[...]
    yT = yT_ref[...]
    zT = zT_ref[...]
    dT = dT_ref[...]
    tjT = tjT_ref[...]
    ti = ti_ref[...]
    w = w_ref[...]
    e = e_ref[...]
    b = b_ref[...]
    c = xT.shape[1]

    w2 = w[2:3, :].reshape(1, 1, 64)
    ew1 = jnp.dot(e, w[3:19, :], preferred_element_type=jnp.float32)
    ew2 = jnp.dot(e, w[19:35, :], preferred_element_type=jnp.float32)
    ew3 = jnp.dot(e, w[35:51, :], preferred_element_type=jnp.float32)
    zpad = jnp.zeros((15, 64), jnp.float32)
    wpa = jnp.concatenate([ew2, w[0:1, :], zpad], axis=0)
    wpb = jnp.concatenate([ew3, w[1:2, :], zpad], axis=0)

    invT = 1.0 / dT
    tj2 = jnp.transpose(tjT).astype(jnp.int32)
    d2 = jnp.transpose(dT)
    inv2 = jnp.transpose(invT)
    iota32 = lax.broadcasted_iota(jnp.int32, (c, _N_NEIGH, 32), 2)
    tj_s = jnp.broadcast_to(tj2[:, :, None], (c, _N_NEIGH, 32))
    inv_s = jnp.broadcast_to(inv2[:, :, None], (c, _N_NEIGH, 32))
    d_s = jnp.broadcast_to(d2[:, :, None], (c, _N_NEIGH, 32))
    x_lhs = jnp.where(tj_s == iota32, inv_s, 0.0)
    x_lhs = jnp.where(iota32 == _N_TYPES, d_s, x_lhs)
    x2 = x_lhs.reshape(c * _N_NEIGH, 32)
    a3 = jnp.dot(x2, wpa, preferred_element_type=jnp.float32).reshape(
        c, _N_NEIGH, 64)
    b3 = jnp.dot(x2, wpb, preferred_element_type=jnp.float32).reshape(
        c, _N_NEIGH, 64)

    iota2 = lax.broadcasted_iota(jnp.int32, (c, _N_TYPES), 1)
    onei = (ti.astype(jnp.int32) == iota2).astype(jnp.float32)
    cc = jnp.dot(onei, ew1, preferred_element_type=jnp.float32) + b
    s3 = a3 + cc.reshape(c, 1, 64)

    rows = []
    for j in range(_N_NEIGH):
        dx = xT[j:j + 1, :] - xT
        dy = yT[j:j + 1, :] - yT
        dz = zT[j:j + 1, :] - zT
        djk = jnp.sqrt(dx * dx + dy * dy + dz * dz)
        mx = jnp.maximum(dT[j:j + 1, :], dT)
        mn = jnp.minimum(dT[j:j + 1, :], dT)
        rows.append((djk - mx + mn) / (2.0 * mn))
    dall = jnp.transpose(jnp.concatenate(rows, axis=0))

    for j in range(_N_NEIGH):
        if j == 0:
            b7 = b3[:, 1:, :]
            dj7 = dall[:, 1:8]
        elif j == _N_NEIGH - 1:
            b7 = b3[:, :j, :]
            dj7 = dall[:, 8 * j:8 * j + j]
        else:
            b7 = jnp.concatenate([b3[:, :j, :], b3[:, j + 1:, :]], axis=1)
            dj7 = jnp.concatenate(
                [dall[:, 8 * j:8 * j + j], dall[:, 8 * j + j + 1:8 * (j + 1)]],
                axis=1)
        out_ref[:, 7 * j:7 * (j + 1), :] = (
            s3[:, j:j + 1, :] + b7 + dj7[:, :, None] * w2)


def kernel(atoms_xyz, atom_type, atom_i_idx, atom_j_idx, dist_ij, embed_table, W, b):
    xf = atoms_xyz[:, 0].astype(jnp.float32)
    yf = atoms_xyz[:, 1].astype(jnp.float32)
    zf = atoms_xyz[:, 2].astype(jnp.float32)
    tf = atom_type.astype(jnp.float32)
    idx_j = jnp.transpose(atom_j_idx).reshape(-1).astype(jnp.int32)
    idx_i = atom_i_idx.reshape(-1).astype(jnp.int32)

    xg, yg, zg, tg, tig = _sc_gather(xf, yf, zf, tf, idx_j, idx_i)
    xT = xg.reshape(_N_NEIGH, _N_CENTER)
    yT = yg.reshape(_N_NEIGH, _N_CENTER)
    zT = zg.reshape(_N_NEIGH, _N_CENTER)
    tjT = tg.reshape(_N_NEIGH, _N_CENTER)
    ti = tig.reshape(_N_CENTER, 1)
    dT = jnp.transpose(dist_ij.astype(jnp.float32))
    b2 = b.astype(jnp.float32).reshape(1, 64)

    grid = _N_CENTER // _CBLK
    out = pl.pallas_call(
        _tc_body,
        grid=(grid,),
        in_specs=[
            pl.BlockSpec((_N_NEIGH, _CBLK), lambda i: (0, i)),
            pl.BlockSpec((_N_NEIGH, _CBLK), lambda i: (0, i)),
            pl.BlockSpec((_N_NEIGH, _CBLK), lambda i: (0, i)),
            pl.BlockSpec((_N_NEIGH, _CBLK), lambda i: (0, i)),
            pl.BlockSpec((_N_NEIGH, _CBLK), lambda i: (0, i)),
            pl.BlockSpec((_CBLK, 1), lambda i: (i, 0)),
            pl.BlockSpec((51, 64), lambda i: (0, 0)),
            pl.BlockSpec((16, 16), lambda i: (0, 0)),
            pl.BlockSpec((1, 64), lambda i: (0, 0)),
        ],
        out_specs=pl.BlockSpec((_CBLK, 56, 64), lambda i: (i, 0, 0)),
        out_shape=jax.ShapeDtypeStruct((_N_CENTER, 56, 64), jnp.float32),
    )(xT, yT, zT, dT, tjT, ti, W.astype(jnp.float32),
      embed_table.astype(jnp.float32), b2)
    return out


# speedup vs baseline: 14.4622x; 4.3539x over previous
import functools

import jax
import jax.numpy as jnp
from jax import lax
from jax.experimental import pallas as pl
from jax.experimental.pallas import tpu as pltpu
from jax.experimental.pallas import tpu_sc as plsc

_N_ATOMS = 100000
_N_CENTER = 8192
_N_NEIGH = 8
_N_TYPES = 16
_NC = 2
_NS = 16
_NW = _NC * _NS
_CBLK = 1024


def _sc_gather(xf, yf, zf, tf, idx_j, idx_i):
    nj = idx_j.shape[0] // _NW
    ni = idx_i.shape[0] // _NW
    mesh = plsc.VectorSubcoreMesh(core_axis_name="c", subcore_axis_name="s")
    n_total = idx_j.shape[0]

    @functools.partial(
        pl.kernel,
        mesh=mesh,
        compiler_params=pltpu.CompilerParams(use_tc_tiling_on_sc=False),
        out_type=[
            jax.ShapeDtypeStruct((n_total,), jnp.float32),
            jax.ShapeDtypeStruct((n_total,), jnp.float32),
            jax.ShapeDtypeStruct((n_total,), jnp.float32),
            jax.ShapeDtypeStruct((n_total,), jnp.int32),
            jax.ShapeDtypeStruct((idx_i.shape[0],), jnp.int32),
        ],
        scratch_types=[
            pltpu.VMEM((nj,), jnp.int32),
            pltpu.VMEM((nj,), jnp.float32),
            pltpu.VMEM((nj,), jnp.float32),
            pltpu.VMEM((nj,), jnp.float32),
            pltpu.VMEM((nj,), jnp.int32),
            pltpu.VMEM((ni,), jnp.int32),
            pltpu.VMEM((ni,), jnp.int32),
            pltpu.SemaphoreType.DMA,
            pltpu.SemaphoreType.DMA,
        ],
    )
    def k(xf_hbm, yf_hbm, zf_hbm, tf_hbm, idxj_hbm, idxi_hbm,
          xg_hbm, yg_hbm, zg_hbm, tg_hbm, tig_hbm,
          idxj_v, xv, yv, zv, tv, idxi_v, tiv, sem, sem2):
        wid = lax.axis_index("s") * _NC + lax.axis_index("c")
        bj = wid * nj
        bi = wid * ni
        pltpu.sync_copy(idxj_hbm.at[pl.ds(bj, nj)], idxj_v)
        pltpu.sync_copy(idxi_hbm.at[pl.ds(bi, ni)], idxi_v)
        cx = pltpu.async_copy(xf_hbm.at[idxj_v], xv, sem)
        cy = pltpu.async_copy(yf_hbm.at[idxj_v], yv, sem)
        cz = pltpu.async_copy(zf_hbm.at[idxj_v], zv, sem)
        ct = pltpu.async_copy(tf_hbm.at[idxj_v], tv, sem)
        ci = pltpu.async_copy(tf_hbm.at[idxi_v], tiv, sem)
        cx.wait()
        ox = pltpu.async_copy(xv, xg_hbm.at[pl.ds(bj, nj)], sem2)
        cy.wait()
        oy = pltpu.async_copy(yv, yg_hbm.at[pl.ds(bj, nj)], sem2)
        cz.wait()
        oz = pltpu.async_copy(zv, zg_hbm.at[pl.ds(bj, nj)], sem2)
        ct.wait()
        ot = pltpu.async_copy(tv, tg_hbm.at[pl.ds(bj, nj)], sem2)
        ci.wait()
        oi = pltpu.async_copy(tiv, tig_hbm.at[pl.ds(bi, ni)], sem2)
        ox.wait()
        oy.wait()
        oz.wait()
        ot.wait()
        oi.wait()

    return k(xf, yf, zf, tf, idx_j, idx_i)


def _tc_body(xT_ref, yT_ref, zT_ref, dT_ref, tjT_ref, tiT_ref,
             w_ref, e_ref, b_ref, out_ref):
    nn = _N_NEIGH
    no = 64
    c = tjT_ref.shape[1] * 128
    xT = xT_ref[...].reshape(nn, c)
    yT = yT_ref[...].reshape(nn, c)
    zT = zT_ref[...].reshape(nn, c)
    dT = dT_ref[...]
    tji = tjT_ref[...].reshape(nn, c)
    tiT = tiT_ref[...].reshape(1, c)
    w = w_ref[...]
    e = e_ref[...]
    b = b_ref[...]

    ew1t = jnp.transpose(jnp.dot(e, w[3:19, :], preferred_element_type=jnp.float32))
    ew2t = jnp.transpose(jnp.dot(e, w[19:35, :], preferred_element_type=jnp.float32))
    ew3t = jnp.transpose(jnp.dot(e, w[35:51, :], preferred_element_type=jnp.float32))
    w0f = jnp.broadcast_to(jnp.transpose(w[0:1, :]), (no, c))
    w1f = jnp.broadcast_to(jnp.transpose(w[1:2, :]), (no, c))
    w2f = jnp.broadcast_to(jnp.transpose(w[2:3, :]), (no, c))
    bf = jnp.broadcast_to(jnp.transpose(b), (no, c))

    iota16 = lax.broadcasted_iota(jnp.int32, (_N_TYPES, c), 0)
    oh = [
        (jnp.broadcast_to(tji[n:n + 1, :], (_N_TYPES, c)) == iota16
         ).astype(jnp.float32)
        for n in range(nn)
    ]
    ohall = jnp.concatenate(oh, axis=1)
    m2t = jnp.dot(ew2t, ohall, preferred_element_type=jnp.float32)
    m3t = jnp.dot(ew3t, ohall, preferred_element_type=jnp.float32)
    ohi = (jnp.broadcast_to(tiT, (_N_TYPES, c)) == iota16
           ).astype(jnp.float32)
    cct = jnp.dot(ew1t, ohi, preferred_element_type=jnp.float32) + bf

    invt = 1.0 / dT
    s_j = []
    b_k = []
    for n in range(nn):
        dbc = jnp.broadcast_to(dT[n:n + 1, :], (no, c))
        ibc = jnp.broadcast_to(invt[n:n + 1, :], (no, c))
        m2n = m2t[:, n * c:(n + 1) * c]
        m3n = m3t[:, n * c:(n + 1) * c]
        s_j.append(cct + w0f * dbc + ibc * m2n)
        b_k.append(w1f * dbc + ibc * m3n)

    for j in range(nn):
        dx = xT[j:j + 1, :] - xT
        dy = yT[j:j + 1, :] - yT
        dz = zT[j:j + 1, :] - zT
        djk = jnp.sqrt(dx * dx + dy * dy + dz * dz)
        mx = jnp.maximum(dT[j:j + 1, :], dT)
        mn = jnp.minimum(dT[j:j + 1, :], dT)
        djkn = (djk - mx + mn) / (2.0 * mn)
        p = j * (nn - 1)
        for k in range(nn):
            if k == j:
                continue
            dbk = jnp.broadcast_to(djkn[k:k + 1, :], (no, c))
            out_ref[p, :, :] = s_j[j] + b_k[k] + w2f * dbk
            p += 1


def kernel(atoms_xyz, atom_type, atom_i_idx, atom_j_idx, dist_ij, embed_table, W, b):
    xf = atoms_xyz[:, 0].astype(jnp.float32)
    yf = atoms_xyz[:, 1].astype(jnp.float32)
    zf = atoms_xyz[:, 2].astype(jnp.float32)
    tf = atom_type.astype(jnp.int32)
    idx_j = jnp.transpose(atom_j_idx).reshape(-1).astype(jnp.int32)
    idx_i = atom_i_idx.reshape(-1).astype(jnp.int32)

    xg, yg, zg, tg, tig = _sc_gather(xf, yf, zf, tf, idx_j, idx_i)
    nb_all = _N_CENTER // 128
    nb = _CBLK // 128
    xT = xg.reshape(_N_NEIGH, nb_all, 128)
    yT = yg.reshape(_N_NEIGH, nb_all, 128)
    zT = zg.reshape(_N_NEIGH, nb_all, 128)
    tjT = tg.reshape(_N_NEIGH, nb_all, 128)
    tiT = tig.reshape(nb_all, 128)
    dT = jnp.transpose(dist_ij.astype(jnp.float32))
    b2 = b.astype(jnp.float32).reshape(1, 64)

    grid = _N_CENTER // _CBLK
    out_t = pl.pallas_call(
        _tc_body,
        grid=(grid,),
        compiler_params=pltpu.CompilerParams(
            vmem_limit_bytes=50 * 1024 * 1024),
        in_specs=[
            pl.BlockSpec((_N_NEIGH, nb, 128), lambda i: (0, i, 0)),
            pl.BlockSpec((_N_NEIGH, nb, 128), lambda i: (0, i, 0)),
            pl.BlockSpec((_N_NEIGH, nb, 128), lambda i: (0, i, 0)),
            pl.BlockSpec((_N_NEIGH, _CBLK), lambda i: (0, i)),
            pl.BlockSpec((_N_NEIGH, nb, 128), lambda i: (0, i, 0)),
            pl.BlockSpec((nb, 128), lambda i: (i, 0)),
            pl.BlockSpec((51, 64), lambda i: (0, 0)),
            pl.BlockSpec((16, 16), lambda i: (0, 0)),
            pl.BlockSpec((1, 64), lambda i: (0, 0)),
        ],
        out_specs=pl.BlockSpec((56, 64, _CBLK), lambda i: (0, 0, i)),
        out_shape=jax.ShapeDtypeStruct((56, 64, _N_CENTER), jnp.float32),
    )(xT, yT, zT, dT, tjT, tiT, W.astype(jnp.float32),
      embed_table.astype(jnp.float32), b2)
    return jnp.transpose(out_t, (2, 0, 1))
